# baseline (speedup 1.0000x reference)
import functools

import jax
import jax.numpy as jnp
from jax import lax
from jax.experimental import pallas as pl
from jax.experimental.pallas import tpu as pltpu
from jax.experimental.pallas import tpu_sc as plsc

VOCAB = 100000
D = 128
DEPTH = 17
N_INTERNAL = 2**DEPTH - 1
B = 4096
CTX = 8

_INFO = plsc.get_sparse_core_info()
NC, NS, L = _INFO.num_cores, _INFO.num_subcores, _INFO.num_lanes
NW = NC * NS
BW = B // NW
NCHUNK = BW // L
NG = BW // L
DV = D // L
TSTRIDE = L + 1

_mesh = plsc.VectorSubcoreMesh(core_axis_name="c", subcore_axis_name="s")


@functools.partial(
    pl.kernel,
    out_type=[
        jax.ShapeDtypeStruct((B,), jnp.int32),
        jax.ShapeDtypeStruct((NW, DEPTH, BW), jnp.float32),
    ],
    mesh=_mesh,
    compiler_params=pltpu.CompilerParams(needs_layout_passes=False),
    scratch_types=[
        pltpu.VMEM((L * CTX, D), jnp.float32),
        pltpu.VMEM((BW, D), jnp.float32),
        pltpu.VMEM((BW, D), jnp.float32),
        pltpu.VMEM((L * CTX,), jnp.int32),
        pltpu.VMEM((BW,), jnp.int32),
        pltpu.VMEM((DEPTH, BW), jnp.float32),
        pltpu.VMEM((L * TSTRIDE,), jnp.float32),
        pltpu.SemaphoreType.DMA,
    ],
)
def _hs_kernel(ctx_hbm, emb_hbm, th_hbm, leaf_hbm, scores_hbm,
               rows_v, xw_v, theta_v, idx_v, nodes_v, scoresT_v, tbuf_v, sem):
    wid = lax.axis_index("s") * NC + lax.axis_index("c")
    base = pl.multiple_of(wid * BW, BW)
    lanes = jnp.arange(L, dtype=jnp.int32)

    def chunk_body(c, carry):
        off = pl.multiple_of((base + c * L) * CTX, L * CTX)
        pltpu.sync_copy(ctx_hbm.at[pl.ds(off, L * CTX)], idx_v)
        pltpu.async_copy(emb_hbm.at[idx_v], rows_v, sem).wait()
        for e in range(L):
            for k in range(DV):
                ds = pl.ds(k * L, L)
                acc = rows_v[e * CTX, ds]
                for j in range(1, CTX):
                    acc = acc + rows_v[e * CTX + j, ds]
                xw_v[c * L + e, ds] = acc
        return carry

    with jax.named_scope("embed_phase"):
        lax.fori_loop(0, NCHUNK, chunk_body, 0)

    for g in range(NG):
        nodes_v[pl.ds(g * L, L)] = jnp.zeros((L,), jnp.int32)

    def step_body(t, carry):
        with jax.named_scope("theta_gather"):
            pltpu.async_copy(th_hbm.at[nodes_v], theta_v, sem).wait()

        def group_body(g, carry2):
            gds = pl.ds(g * L, L)
            e0 = g * L
            for e in range(L):
                p = []
                for k in range(DV):
                    ds = pl.ds(k * L, L)
                    p.append(xw_v[e0 + e, ds] * theta_v[e0 + e, ds])
                acc = ((p[0] + p[1]) + (p[2] + p[3])) + (
                    (p[4] + p[5]) + (p[6] + p[7]))
                plsc.store_scatter(tbuf_v, [lanes * TSTRIDE + e], acc)
            score = tbuf_v[pl.ds(0, L)]
            for r in range(1, L):
                score = score + tbuf_v[pl.ds(r * TSTRIDE, L)]
            scoresT_v[t, gds] = score
            nd = nodes_v[gds]
            nodes_v[gds] = 2 * nd + jnp.where(score < 0.0, 1, 2)
            return carry2

        with jax.named_scope("step_compute"):
            return lax.fori_loop(0, NG, group_body, carry)

    lax.fori_loop(0, DEPTH, step_body, 0)

    for g in range(NG):
        gds = pl.ds(g * L, L)
        nodes_v[gds] = nodes_v[gds] - N_INTERNAL
    pltpu.sync_copy(nodes_v, leaf_hbm.at[pl.ds(base, BW)])
    pltpu.sync_copy(scoresT_v, scores_hbm.at[wid])


def kernel(context_vector, embeddings, thetas):
    ctx_flat = context_vector.reshape(-1).astype(jnp.int32)
    leaf, scores3 = _hs_kernel(ctx_flat, embeddings, thetas)
    scores = scores3.transpose(0, 2, 1).reshape(B, DEPTH)
    return leaf, scores


# speedup vs baseline: 2.7051x; 2.7051x over previous
import functools

import jax
import jax.numpy as jnp
from jax import lax
from jax.experimental import pallas as pl
from jax.experimental.pallas import tpu as pltpu
from jax.experimental.pallas import tpu_sc as plsc

VOCAB = 100000
D = 128
DEPTH = 17
N_INTERNAL = 2**DEPTH - 1
B = 4096
CTX = 8

_INFO = plsc.get_sparse_core_info()
NC, NS, L = _INFO.num_cores, _INFO.num_subcores, _INFO.num_lanes
NW = NC * NS
BW = B // NW
NCHUNK = BW // L
NG = BW // L
DV = D // L
TSTRIDE = L + 1
TPRE = 9
NPRE = 2**TPRE

_mesh = plsc.VectorSubcoreMesh(core_axis_name="c", subcore_axis_name="s")


@functools.partial(
    pl.kernel,
    out_type=[
        jax.ShapeDtypeStruct((B,), jnp.int32),
        jax.ShapeDtypeStruct((NW, DEPTH, BW), jnp.float32),
    ],
    mesh=_mesh,
    compiler_params=pltpu.CompilerParams(needs_layout_passes=False),
    scratch_types=[
        pltpu.VMEM((L * CTX, D), jnp.float32),
        pltpu.VMEM((BW, D), jnp.float32),
        pltpu.VMEM((BW, D), jnp.float32),
        pltpu.VMEM((NPRE, D), jnp.float32),
        pltpu.VMEM((L * CTX,), jnp.int32),
        pltpu.VMEM((BW,), jnp.int32),
        pltpu.VMEM((DEPTH, BW), jnp.float32),
        pltpu.VMEM((L * TSTRIDE,), jnp.float32),
        pltpu.SemaphoreType.DMA,
        pltpu.SemaphoreType.DMA,
    ],
)
def _hs_kernel(ctx_hbm, emb_hbm, th_hbm, leaf_hbm, scores_hbm,
               rows_v, xw_v, theta_v, top_v, idx_v, nodes_v, scoresT_v,
               tbuf_v, sem, sem2):
    wid = lax.axis_index("s") * NC + lax.axis_index("c")
    base = pl.multiple_of(wid * BW, BW)
    lanes = jnp.arange(L, dtype=jnp.int32)

    top_cp = pltpu.async_copy(th_hbm.at[pl.ds(0, NPRE)], top_v, sem2)

    def chunk_body(c, carry):
        off = pl.multiple_of((base + c * L) * CTX, L * CTX)
        pltpu.sync_copy(ctx_hbm.at[pl.ds(off, L * CTX)], idx_v)
        pltpu.async_copy(emb_hbm.at[idx_v], rows_v, sem).wait()
        for e in range(L):
            for k in range(DV):
                ds = pl.ds(k * L, L)
                acc = rows_v[e * CTX, ds]
                for j in range(1, CTX):
                    acc = acc + rows_v[e * CTX + j, ds]
                xw_v[c * L + e, ds] = acc
        return carry

    with jax.named_scope("embed_phase"):
        lax.fori_loop(0, NCHUNK, chunk_body, 0)

    for g in range(NG):
        nodes_v[pl.ds(g * L, L)] = jnp.zeros((L,), jnp.int32)

    def make_group_body(t, theta_ref, local_nodes):
        def group_body(g, carry2):
            gds = pl.ds(g * L, L)
            e0 = g * L
            nd_vec = nodes_v[gds]
            for e in range(L):
                if local_nodes:
                    row = nd_vec[e]
                else:
                    row = e0 + e
                p = []
                for k in range(DV):
                    ds = pl.ds(k * L, L)
                    p.append(xw_v[e0 + e, ds] * theta_ref[row, ds])
                acc = ((p[0] + p[1]) + (p[2] + p[3])) + (
                    (p[4] + p[5]) + (p[6] + p[7]))
                plsc.store_scatter(tbuf_v, [lanes * TSTRIDE + e], acc)
            score = tbuf_v[pl.ds(0, L)]
            for r in range(1, L):
                score = score + tbuf_v[pl.ds(r * TSTRIDE, L)]
            scoresT_v[t, gds] = score
            nd = nodes_v[gds]
            nodes_v[gds] = 2 * nd + jnp.where(score < 0.0, 1, 2)
            return carry2

        return group_body

    with jax.named_scope("top_wait"):
        top_cp.wait()

    def pre_step_body(t, carry):
        return lax.fori_loop(0, NG, make_group_body(t, top_v, True), carry)

    with jax.named_scope("pre_steps"):
        lax.fori_loop(0, TPRE, pre_step_body, 0)

    def step_body(t, carry):
        with jax.named_scope("theta_gather"):
            pltpu.async_copy(th_hbm.at[nodes_v], theta_v, sem).wait()
        with jax.named_scope("step_compute"):
            return lax.fori_loop(0, NG, make_group_body(t, theta_v, False),
                                 carry)

    lax.fori_loop(TPRE, DEPTH, step_body, 0)

    for g in range(NG):
        gds = pl.ds(g * L, L)
        nodes_v[gds] = nodes_v[gds] - N_INTERNAL
    pltpu.sync_copy(nodes_v, leaf_hbm.at[pl.ds(base, BW)])
    pltpu.sync_copy(scoresT_v, scores_hbm.at[wid])


def kernel(context_vector, embeddings, thetas):
    ctx_flat = context_vector.reshape(-1).astype(jnp.int32)
    leaf, scores3 = _hs_kernel(ctx_flat, embeddings, thetas)
    scores = scores3.transpose(0, 2, 1).reshape(B, DEPTH)
    return leaf, scores


# speedup vs baseline: 3.2816x; 1.2131x over previous
import functools

import jax
import jax.numpy as jnp
from jax import lax
from jax.experimental import pallas as pl
from jax.experimental.pallas import tpu as pltpu
from jax.experimental.pallas import tpu_sc as plsc

VOCAB = 100000
D = 128
DEPTH = 17
N_INTERNAL = 2**DEPTH - 1
B = 4096
CTX = 8

_INFO = plsc.get_sparse_core_info()
NC, NS, L = _INFO.num_cores, _INFO.num_subcores, _INFO.num_lanes
NW = NC * NS
BW = B // NW
NCHUNK = BW // L
NG = BW // L
DV = D // L
TSTRIDE = L + 1
TPRE = 9
NPRE = 2**TPRE

_mesh = plsc.VectorSubcoreMesh(core_axis_name="c", subcore_axis_name="s")


@functools.partial(
    pl.kernel,
    out_type=[
        jax.ShapeDtypeStruct((B,), jnp.int32),
        jax.ShapeDtypeStruct((B * DEPTH,), jnp.float32),
    ],
    mesh=_mesh,
    compiler_params=pltpu.CompilerParams(needs_layout_passes=False),
    scratch_types=[
        pltpu.VMEM((L * CTX, D), jnp.float32),
        pltpu.VMEM((L * CTX, D), jnp.float32),
        pltpu.VMEM((BW, D), jnp.float32),
        pltpu.VMEM((NPRE, D), jnp.float32),
        pltpu.VMEM((L * CTX,), jnp.int32),
        pltpu.VMEM((L * CTX,), jnp.int32),
        pltpu.VMEM((BW,), jnp.int32),
        pltpu.VMEM((BW * DEPTH,), jnp.float32),
        pltpu.VMEM((L * TSTRIDE,), jnp.float32),
        pltpu.SemaphoreType.DMA,
        pltpu.SemaphoreType.DMA,
        pltpu.SemaphoreType.DMA,
    ],
)
def _hs_kernel(ctx_hbm, emb_hbm, th_hbm, leaf_hbm, scores_hbm,
               rows_a, rows_b, xw_v, top_v, idx_a, idx_b,
               nodes_v, scores_v, tbuf_v, sem_a, sem_b, sem2):
    theta_v = rows_b
    wid = lax.axis_index("s") * NC + lax.axis_index("c")
    base = pl.multiple_of(wid * BW, BW)
    lanes = jnp.arange(L, dtype=jnp.int32)

    top_cp = pltpu.async_copy(th_hbm.at[pl.ds(0, NPRE)], top_v, sem2)

    def start_gather(c, idx_ref, rows_ref, sem):
        off = pl.multiple_of((base + c * L) * CTX, L * CTX)
        pltpu.sync_copy(ctx_hbm.at[pl.ds(off, L * CTX)], idx_ref)
        pltpu.async_copy(emb_hbm.at[idx_ref], rows_ref, sem)

    def wait_gather(idx_ref, rows_ref, sem):
        pltpu.make_async_copy(emb_hbm.at[idx_ref], rows_ref, sem).wait()

    def sum_chunk(rows_ref, c):
        def elem_body(e, carry):
            for k in range(DV):
                ds = pl.ds(k * L, L)
                acc = rows_ref[e * CTX, ds]
                for j in range(1, CTX):
                    acc = acc + rows_ref[e * CTX + j, ds]
                xw_v[c * L + e, ds] = acc
            return carry

        lax.fori_loop(0, L, elem_body, 0)

    with jax.named_scope("embed_phase"):
        start_gather(0, idx_a, rows_a, sem_a)
        start_gather(1, idx_b, rows_b, sem_b)

        def super_body(s, carry):
            ca = 2 * s
            wait_gather(idx_a, rows_a, sem_a)
            sum_chunk(rows_a, ca)

            @pl.when(s < NCHUNK // 2 - 1)
            def _():
                start_gather(ca + 2, idx_a, rows_a, sem_a)

            wait_gather(idx_b, rows_b, sem_b)
            sum_chunk(rows_b, ca + 1)

            @pl.when(s < NCHUNK // 2 - 1)
            def _():
                start_gather(ca + 3, idx_b, rows_b, sem_b)

            return carry

        lax.fori_loop(0, NCHUNK // 2, super_body, 0)

    for g in range(NG):
        nodes_v[pl.ds(g * L, L)] = jnp.zeros((L,), jnp.int32)

    def finish_group(g, t, score):
        gds = pl.ds(g * L, L)
        plsc.store_scatter(scores_v, [(g * L + lanes) * DEPTH + t], score)
        nd = nodes_v[gds]
        nodes_v[gds] = 2 * nd + jnp.where(score < 0.0, 1, 2)

    def transpose_reduce():
        score = tbuf_v[pl.ds(0, L)]
        for r in range(1, L):
            score = score + tbuf_v[pl.ds(r * TSTRIDE, L)]
        return score

    def make_group_body(t, theta_ref, local_nodes):
        def group_body(g, carry2):
            e0 = g * L
            nd_vec = nodes_v[pl.ds(e0, L)]
            for e in range(L):
                row = nd_vec[e] if local_nodes else e0 + e
                p = []
                for k in range(DV):
                    ds = pl.ds(k * L, L)
                    p.append(xw_v[e0 + e, ds] * theta_ref[row, ds])
                acc = ((p[0] + p[1]) + (p[2] + p[3])) + (
                    (p[4] + p[5]) + (p[6] + p[7]))
                plsc.store_scatter(tbuf_v, [lanes * TSTRIDE + e], acc)
            finish_group(g, t, transpose_reduce())
            return carry2

        return group_body

    with jax.named_scope("top_wait"):
        top_cp.wait()

    def level0_body(g, carry):
        e0 = g * L
        th0 = [top_v[0, pl.ds(k * L, L)] for k in range(DV)]
        for e in range(L):
            p = [xw_v[e0 + e, pl.ds(k * L, L)] * th0[k] for k in range(DV)]
            acc = ((p[0] + p[1]) + (p[2] + p[3])) + (
                (p[4] + p[5]) + (p[6] + p[7]))
            plsc.store_scatter(tbuf_v, [lanes * TSTRIDE + e], acc)
        score = transpose_reduce()
        plsc.store_scatter(scores_v, [(e0 + lanes) * DEPTH], score)
        nodes_v[pl.ds(e0, L)] = jnp.where(score < 0.0, 1, 2)
        return carry

    with jax.named_scope("pre_steps"):
        lax.fori_loop(0, NG, level0_body, 0)

        def pre_step_body(t, carry):
            return lax.fori_loop(0, NG, make_group_body(t, top_v, True),
                                 carry)

        lax.fori_loop(1, TPRE, pre_step_body, 0)

    def step_body(t, carry):
        with jax.named_scope("theta_gather"):
            pltpu.async_copy(th_hbm.at[nodes_v], theta_v, sem_a).wait()
        with jax.named_scope("step_compute"):
            return lax.fori_loop(0, NG, make_group_body(t, theta_v, False),
                                 carry)

    lax.fori_loop(TPRE, DEPTH, step_body, 0)

    for g in range(NG):
        gds = pl.ds(g * L, L)
        nodes_v[gds] = nodes_v[gds] - N_INTERNAL
    pltpu.sync_copy(nodes_v, leaf_hbm.at[pl.ds(base, BW)])
    pltpu.sync_copy(scores_v,
                    scores_hbm.at[pl.ds(base * DEPTH, BW * DEPTH)])


def kernel(context_vector, embeddings, thetas):
    ctx_flat = context_vector.reshape(-1).astype(jnp.int32)
    leaf, scores_flat = _hs_kernel(ctx_flat, embeddings, thetas)
    return leaf, scores_flat.reshape(B, DEPTH)


# speedup vs baseline: 3.3745x; 1.0283x over previous
import functools

import jax
import jax.numpy as jnp
from jax import lax
from jax.experimental import pallas as pl
from jax.experimental.pallas import tpu as pltpu
from jax.experimental.pallas import tpu_sc as plsc

VOCAB = 100000
D = 128
DEPTH = 17
N_INTERNAL = 2**DEPTH - 1
B = 4096
CTX = 8

_INFO = plsc.get_sparse_core_info()
NC, NS, L = _INFO.num_cores, _INFO.num_subcores, _INFO.num_lanes
NW = NC * NS
BW = B // NW
NG = BW // L
DV = D // L
TSTRIDE = L + 1
TPRE = 9
NPRE = 2**TPRE
HALF = BW // 2

_mesh = plsc.VectorSubcoreMesh(core_axis_name="c", subcore_axis_name="s")


@functools.partial(
    pl.kernel,
    out_type=[
        jax.ShapeDtypeStruct((B,), jnp.int32),
        jax.ShapeDtypeStruct((B * DEPTH,), jnp.float32),
    ],
    mesh=_mesh,
    compiler_params=pltpu.CompilerParams(needs_layout_passes=False),
    scratch_types=[
        pltpu.VMEM((L * CTX, D), jnp.float32),
        pltpu.VMEM((L * CTX, D), jnp.float32),
        pltpu.VMEM((BW, D), jnp.float32),
        pltpu.VMEM((NPRE, D), jnp.float32),
        pltpu.VMEM((L * CTX,), jnp.int32),
        pltpu.VMEM((L * CTX,), jnp.int32),
        pltpu.VMEM((BW,), jnp.int32),
        pltpu.VMEM((BW * DEPTH,), jnp.float32),
        pltpu.VMEM((L * TSTRIDE,), jnp.float32),
        pltpu.SemaphoreType.DMA,
        pltpu.SemaphoreType.DMA,
        pltpu.SemaphoreType.DMA,
    ],
)
def _hs_kernel(ctx_hbm, emb_hbm, th_hbm, leaf_hbm, scores_hbm,
               rows_a, rows_b, xw_v, top_v, idx_a, idx_b,
               nodes_v, scores_v, tbuf_v, sem_a, sem_b, sem2):
    wid = lax.axis_index("s") * NC + lax.axis_index("c")
    base = pl.multiple_of(wid * BW, BW)
    lanes = jnp.arange(L, dtype=jnp.int32)

    top_cp = pltpu.async_copy(th_hbm.at[pl.ds(0, NPRE)], top_v, sem2)

    def start_gather(c, idx_ref, rows_ref, sem):
        off = pl.multiple_of((base + c * L) * CTX, L * CTX)
        pltpu.sync_copy(ctx_hbm.at[pl.ds(off, L * CTX)], idx_ref)
        pltpu.async_copy(emb_hbm.at[idx_ref], rows_ref, sem)

    def wait_gather(idx_ref, rows_ref, sem):
        pltpu.make_async_copy(emb_hbm.at[idx_ref], rows_ref, sem).wait()

    def sum_chunk(rows_ref, g):
        def elem_body(e, carry):
            for k in range(DV):
                ds = pl.ds(k * L, L)
                acc = rows_ref[e * CTX, ds]
                for j in range(1, CTX):
                    acc = acc + rows_ref[e * CTX + j, ds]
                xw_v[g * L + e, ds] = acc
            return carry

        lax.fori_loop(0, L, elem_body, 0)

    def transpose_reduce():
        score = tbuf_v[pl.ds(0, L)]
        for r in range(1, L):
            score = score + tbuf_v[pl.ds(r * TSTRIDE, L)]
        return score

    def finish_group(g, t, score):
        gds = pl.ds(g * L, L)
        plsc.store_scatter(scores_v, [(g * L + lanes) * DEPTH + t], score)
        nd = nodes_v[gds]
        nodes_v[gds] = 2 * nd + jnp.where(score < 0.0, 1, 2)

    def group_level(g, t, theta_ref, local_nodes, row_off=0):
        e0 = g * L
        nd_vec = nodes_v[pl.ds(e0, L)]
        for e in range(L):
            row = nd_vec[e] if local_nodes else e0 + e - row_off
            p = []
            for k in range(DV):
                ds = pl.ds(k * L, L)
                p.append(xw_v[e0 + e, ds] * theta_ref[row, ds])
            acc = ((p[0] + p[1]) + (p[2] + p[3])) + (
                (p[4] + p[5]) + (p[6] + p[7]))
            plsc.store_scatter(tbuf_v, [lanes * TSTRIDE + e], acc)
        finish_group(g, t, transpose_reduce())

    def level0_group(g):
        e0 = g * L
        th0 = [top_v[0, pl.ds(k * L, L)] for k in range(DV)]
        for e in range(L):
            p = [xw_v[e0 + e, pl.ds(k * L, L)] * th0[k] for k in range(DV)]
            acc = ((p[0] + p[1]) + (p[2] + p[3])) + (
                (p[4] + p[5]) + (p[6] + p[7]))
            plsc.store_scatter(tbuf_v, [lanes * TSTRIDE + e], acc)
        score = transpose_reduce()
        plsc.store_scatter(scores_v, [(e0 + lanes) * DEPTH], score)
        nodes_v[pl.ds(e0, L)] = jnp.where(score < 0.0, 1, 2)

    def pre_traverse(g):
        level0_group(g)

        def t_body(t, carry):
            group_level(g, t, top_v, True)
            return carry

        lax.fori_loop(1, TPRE, t_body, 0)

    with jax.named_scope("embed_pre"):
        start_gather(0, idx_a, rows_a, sem_a)
        start_gather(1, idx_b, rows_b, sem_b)
        top_cp.wait()

        def super_body(s, carry):
            ga = 2 * s
            wait_gather(idx_a, rows_a, sem_a)
            sum_chunk(rows_a, ga)

            @pl.when(s < NG // 2 - 1)
            def _():
                start_gather(ga + 2, idx_a, rows_a, sem_a)

            pre_traverse(ga)

            wait_gather(idx_b, rows_b, sem_b)
            sum_chunk(rows_b, ga + 1)

            @pl.when(s < NG // 2 - 1)
            def _():
                start_gather(ga + 3, idx_b, rows_b, sem_b)

            pre_traverse(ga + 1)
            return carry

        lax.fori_loop(0, NG // 2, super_body, 0)

    nodes_lo = nodes_v.at[pl.ds(0, HALF)]
    nodes_hi = nodes_v.at[pl.ds(HALF, HALF)]
    th_lo = rows_a.at[pl.ds(0, HALF)]
    th_hi = rows_b.at[pl.ds(0, HALF)]

    def start_lo():
        pltpu.async_copy(th_hbm.at[nodes_lo], th_lo, sem_a)

    def start_hi():
        pltpu.async_copy(th_hbm.at[nodes_hi], th_hi, sem_b)

    def wait_lo():
        pltpu.make_async_copy(th_hbm.at[nodes_lo], th_lo, sem_a).wait()

    def wait_hi():
        pltpu.make_async_copy(th_hbm.at[nodes_hi], th_hi, sem_b).wait()

    with jax.named_scope("deep_steps"):
        start_lo()
        start_hi()

        def step_body(t, carry):
            wait_lo()

            def glo_body(g, c2):
                group_level(g, t, rows_a, False, row_off=0)
                return c2

            lax.fori_loop(0, NG // 2, glo_body, 0)

            @pl.when(t < DEPTH - 1)
            def _():
                start_lo()

            wait_hi()

            def ghi_body(g, c2):
                group_level(g, t, rows_b, False, row_off=HALF)
                return c2

            lax.fori_loop(NG // 2, NG, ghi_body, 0)

            @pl.when(t < DEPTH - 1)
            def _():
                start_hi()

            return carry

        lax.fori_loop(TPRE, DEPTH, step_body, 0)

    for g in range(NG):
        gds = pl.ds(g * L, L)
        nodes_v[gds] = nodes_v[gds] - N_INTERNAL
    pltpu.sync_copy(nodes_v, leaf_hbm.at[pl.ds(base, BW)])
    pltpu.sync_copy(scores_v,
                    scores_hbm.at[pl.ds(base * DEPTH, BW * DEPTH)])


def kernel(context_vector, embeddings, thetas):
    ctx_flat = context_vector.reshape(-1).astype(jnp.int32)
    leaf, scores_flat = _hs_kernel(ctx_flat, embeddings, thetas)
    return leaf, scores_flat.reshape(B, DEPTH)


# speedup vs baseline: 3.4696x; 1.0282x over previous
import functools

import jax
import jax.numpy as jnp
from jax import lax
from jax.experimental import pallas as pl
from jax.experimental.pallas import tpu as pltpu
from jax.experimental.pallas import tpu_sc as plsc

VOCAB = 100000
D = 128
DEPTH = 17
N_INTERNAL = 2**DEPTH - 1
B = 4096
CTX = 8

_INFO = plsc.get_sparse_core_info()
NC, NS, L = _INFO.num_cores, _INFO.num_subcores, _INFO.num_lanes
NW = NC * NS
BW = B // NW
NG = BW // L
DV = D // L
TSTRIDE = L + 1
TPRE = 9
NPRE = 2**TPRE
HALF = BW // 2

_mesh = plsc.VectorSubcoreMesh(core_axis_name="c", subcore_axis_name="s")


@functools.partial(
    pl.kernel,
    out_type=[
        jax.ShapeDtypeStruct((B,), jnp.int32),
        jax.ShapeDtypeStruct((B * DEPTH,), jnp.float32),
    ],
    mesh=_mesh,
    compiler_params=pltpu.CompilerParams(needs_layout_passes=False),
    scratch_types=[
        pltpu.VMEM((L * CTX, D), jnp.float32),
        pltpu.VMEM((L * CTX, D), jnp.float32),
        pltpu.VMEM((BW, D), jnp.float32),
        pltpu.VMEM((NPRE, D), jnp.float32),
        pltpu.VMEM((L * CTX,), jnp.int32),
        pltpu.VMEM((L * CTX,), jnp.int32),
        pltpu.VMEM((BW,), jnp.int32),
        pltpu.VMEM((BW * DEPTH,), jnp.float32),
        pltpu.VMEM((BW // 2, D), jnp.float32),
        pltpu.VMEM((L * TSTRIDE,), jnp.float32),
        pltpu.SemaphoreType.DMA,
        pltpu.SemaphoreType.DMA,
        pltpu.SemaphoreType.DMA,
    ],
)
def _hs_kernel(ctx_hbm, emb_hbm, th_hbm, leaf_hbm, scores_hbm,
               rows_a, rows_b, xw_v, top_v, idx_a, idx_b,
               nodes_v, scores_v, thlo_v, tbuf_v, sem_a, sem_b, sem2):
    wid = lax.axis_index("s") * NC + lax.axis_index("c")
    base = pl.multiple_of(wid * BW, BW)
    lanes = jnp.arange(L, dtype=jnp.int32)

    top_cp = pltpu.async_copy(th_hbm.at[pl.ds(0, NPRE)], top_v, sem2)

    def start_gather(c, idx_ref, rows_ref, sem):
        off = pl.multiple_of((base + c * L) * CTX, L * CTX)
        pltpu.sync_copy(ctx_hbm.at[pl.ds(off, L * CTX)], idx_ref)
        pltpu.async_copy(emb_hbm.at[idx_ref], rows_ref, sem)

    def wait_gather(idx_ref, rows_ref, sem):
        pltpu.make_async_copy(emb_hbm.at[idx_ref], rows_ref, sem).wait()

    def sum_chunk(rows_ref, g):
        def elem_body(e, carry):
            for k in range(DV):
                ds = pl.ds(k * L, L)
                acc = rows_ref[e * CTX, ds]
                for j in range(1, CTX):
                    acc = acc + rows_ref[e * CTX + j, ds]
                xw_v[g * L + e, ds] = acc
            return carry

        lax.fori_loop(0, L, elem_body, 0)

    def transpose_reduce():
        score = tbuf_v[pl.ds(0, L)]
        for r in range(1, L):
            score = score + tbuf_v[pl.ds(r * TSTRIDE, L)]
        return score

    def finish_group(g, t, score):
        gds = pl.ds(g * L, L)
        plsc.store_scatter(scores_v, [(g * L + lanes) * DEPTH + t], score)
        nd = nodes_v[gds]
        nodes_v[gds] = 2 * nd + jnp.where(score < 0.0, 1, 2)

    def group_level(g, t, theta_ref, local_nodes, row_off=0):
        e0 = g * L
        if local_nodes:
            nd_vec = nodes_v[pl.ds(e0, L)]
        for e in range(L):
            row = nd_vec[e] if local_nodes else e0 + e - row_off
            p = []
            for k in range(DV):
                ds = pl.ds(k * L, L)
                p.append(xw_v[e0 + e, ds] * theta_ref[row, ds])
            acc = ((p[0] + p[1]) + (p[2] + p[3])) + (
                (p[4] + p[5]) + (p[6] + p[7]))
            plsc.store_scatter(tbuf_v, [lanes * TSTRIDE + e], acc)
        finish_group(g, t, transpose_reduce())

    def level0_group(g):
        e0 = g * L
        th0 = [top_v[0, pl.ds(k * L, L)] for k in range(DV)]
        for e in range(L):
            p = [xw_v[e0 + e, pl.ds(k * L, L)] * th0[k] for k in range(DV)]
            acc = ((p[0] + p[1]) + (p[2] + p[3])) + (
                (p[4] + p[5]) + (p[6] + p[7]))
            plsc.store_scatter(tbuf_v, [lanes * TSTRIDE + e], acc)
        score = transpose_reduce()
        plsc.store_scatter(scores_v, [(e0 + lanes) * DEPTH], score)
        nodes_v[pl.ds(e0, L)] = jnp.where(score < 0.0, 1, 2)

    def pre_traverse(g):
        level0_group(g)

        def t_body(t, carry):
            group_level(g, t, top_v, True)
            return carry

        lax.fori_loop(1, TPRE, t_body, 0)

    nodes_lo = nodes_v.at[pl.ds(0, HALF)]
    nodes_hi = nodes_v.at[pl.ds(HALF, HALF)]
    th_lo = thlo_v
    th_hi = rows_b.at[pl.ds(0, HALF)]

    def start_lo():
        pltpu.async_copy(th_hbm.at[nodes_lo], th_lo, sem2)

    def start_hi():
        pltpu.async_copy(th_hbm.at[nodes_hi], th_hi, sem_b)

    def wait_lo():
        pltpu.make_async_copy(th_hbm.at[nodes_lo], th_lo, sem2).wait()

    def wait_hi():
        pltpu.make_async_copy(th_hbm.at[nodes_hi], th_hi, sem_b).wait()

    with jax.named_scope("embed_pre"):
        start_gather(0, idx_a, rows_a, sem_a)
        start_gather(1, idx_b, rows_b, sem_b)

        def super_body(s, carry):
            ga = 2 * s
            wait_gather(idx_a, rows_a, sem_a)
            sum_chunk(rows_a, ga)

            @pl.when(s == 0)
            def _():
                top_cp.wait()

            @pl.when(s < NG // 2 - 1)
            def _():
                start_gather(ga + 2, idx_a, rows_a, sem_a)

            pre_traverse(ga)

            wait_gather(idx_b, rows_b, sem_b)
            sum_chunk(rows_b, ga + 1)

            @pl.when(s < NG // 2 - 1)
            def _():
                start_gather(ga + 3, idx_b, rows_b, sem_b)

            pre_traverse(ga + 1)

            @pl.when(s == 1)
            def _():
                start_lo()

            return carry

        lax.fori_loop(0, NG // 2, super_body, 0)

    with jax.named_scope("deep_steps"):
        start_hi()

        def step_body(t, carry):
            wait_lo()

            def glo_body(g, c2):
                group_level(g, t, thlo_v, False, row_off=0)
                return c2

            lax.fori_loop(0, NG // 2, glo_body, 0)

            @pl.when(t < DEPTH - 1)
            def _():
                start_lo()

            wait_hi()

            def ghi_body(g, c2):
                group_level(g, t, rows_b, False, row_off=HALF)
                return c2

            lax.fori_loop(NG // 2, NG, ghi_body, 0)

            @pl.when(t < DEPTH - 1)
            def _():
                start_hi()

            return carry

        lax.fori_loop(TPRE, DEPTH, step_body, 0)

    for g in range(NG):
        gds = pl.ds(g * L, L)
        nodes_v[gds] = nodes_v[gds] - N_INTERNAL
    pltpu.sync_copy(nodes_v, leaf_hbm.at[pl.ds(base, BW)])
    pltpu.sync_copy(scores_v,
                    scores_hbm.at[pl.ds(base * DEPTH, BW * DEPTH)])


def kernel(context_vector, embeddings, thetas):
    ctx_flat = context_vector.reshape(-1).astype(jnp.int32)
    leaf, scores_flat = _hs_kernel(ctx_flat, embeddings, thetas)
    return leaf, scores_flat.reshape(B, DEPTH)
